# baseline (speedup 1.0000x reference)
import functools

import jax
import jax.numpy as jnp
import numpy as np
from jax import lax
from jax.experimental import pallas as pl
from jax.experimental.pallas import tpu as pltpu
from jax.experimental.pallas import tpu_sc as plsc

VOCAB = 28996
DIM = 768
SEQ = 64
BATCH = 4096

NUM_CORES = 2
NUM_SUBCORES = 16
NUM_WORKERS = NUM_CORES * NUM_SUBCORES

B_TOTAL = BATCH * SEQ
POS_PER_W = SEQ // NUM_WORKERS
CHUNK = 32
N_CHUNKS = BATCH // CHUNK
LANES = 16
COLS = DIM // LANES
PAIRS = DIM // (2 * LANES)
SUB = CHUNK // LANES

_blk = np.arange(LANES)
_PERM = np.concatenate(
    [np.stack([_blk, _blk + LANES], axis=1).reshape(-1) + 2 * LANES * c
     for c in range(PAIRS)])


def _build_kernel():
    mesh = plsc.VectorSubcoreMesh(core_axis_name="c", subcore_axis_name="s")

    @functools.partial(
        pl.kernel,
        mesh=mesh,
        out_type=jax.ShapeDtypeStruct((B_TOTAL, DIM), jnp.float32),
        scratch_types=[
            pltpu.VMEM((POS_PER_W * BATCH,), jnp.int32),
            pltpu.VMEM((POS_PER_W, DIM), jnp.float32),
            pltpu.VMEM((CHUNK, DIM // 2), jnp.int32),
            pltpu.VMEM((CHUNK, DIM // 2), jnp.int32),
            pltpu.VMEM((CHUNK, DIM), jnp.float32),
            pltpu.VMEM((CHUNK, DIM), jnp.float32),
            pltpu.SemaphoreType.DMA,
            pltpu.SemaphoreType.DMA,
            pltpu.SemaphoreType.DMA,
            pltpu.SemaphoreType.DMA,
        ],
    )
    def emb_kernel(x_hbm, table_hbm, pos_hbm, out_hbm,
                   idx_v, pos_v, gbuf0, gbuf1, sbuf0, sbuf1,
                   gsem0, gsem1, ssem0, ssem1):
        wid = lax.axis_index("s") * NUM_CORES + lax.axis_index("c")
        pltpu.sync_copy(x_hbm.at[wid], idx_v)
        pltpu.sync_copy(pos_hbm.at[pl.ds(POS_PER_W * wid, POS_PER_W)], pos_v)

        gbufs = (gbuf0, gbuf1)
        sbufs = (sbuf0, sbuf1)
        gsems = (gsem0, gsem1)
        ssems = (ssem0, ssem1)
        iota = lax.iota(jnp.int32, LANES)

        for l in range(POS_PER_W):
            labs = POS_PER_W * wid + l

            def idx_ref(j, _l=l):
                return idx_v.at[pl.ds(_l * BATCH + j * CHUNK, CHUNK)]

            def scat_idx(j, s, _labs=labs):
                return iota * SEQ + ((j * CHUNK + s * LANES) * SEQ + _labs)

            pvecs = [pos_v[l, pl.ds(c * LANES, LANES)] for c in range(COLS)]

            pltpu.async_copy(table_hbm.at[idx_ref(0)], gbufs[0], gsems[0])
            pltpu.async_copy(table_hbm.at[idx_ref(1)], gbufs[1], gsems[1])

            def pair_body(jo, carry):
                for b in range(2):
                    j = 2 * jo + b
                    gbuf, sbuf = gbufs[b], sbufs[b]
                    gsem, ssem = gsems[b], ssems[b]

                    pltpu.make_async_copy(
                        table_hbm.at[idx_ref(j)], gbuf, gsem).wait()

                    @pl.when(jo >= 1)
                    def _():
                        for s in range(SUB):
                            pltpu.make_async_copy(
                                sbuf.at[pl.ds(s * LANES, LANES)],
                                out_hbm.at[scat_idx(j, s)], ssem).wait()

                    def row_body(r, c2):
                        for c in range(PAIRS):
                            w = gbuf[r, pl.ds(c * LANES, LANES)]
                            lo = lax.bitcast_convert_type(
                                w << 16, jnp.float32)
                            hi = lax.bitcast_convert_type(
                                w & jnp.int32(-65536), jnp.float32)
                            sl_lo = pl.ds(2 * c * LANES, LANES)
                            sl_hi = pl.ds((2 * c + 1) * LANES, LANES)
                            sbuf[r, sl_lo] = lo + pvecs[2 * c]
                            sbuf[r, sl_hi] = hi + pvecs[2 * c + 1]
                        return c2

                    lax.fori_loop(0, CHUNK, row_body, 0, unroll=False)

                    jn = jnp.minimum(j + 2, N_CHUNKS - 2 + b)
                    pltpu.async_copy(table_hbm.at[idx_ref(jn)], gbuf, gsem)

                    for s in range(SUB):
                        pltpu.async_copy(
                            sbuf.at[pl.ds(s * LANES, LANES)],
                            out_hbm.at[scat_idx(j, s)], ssem)
                return carry

            lax.fori_loop(0, N_CHUNKS // 2, pair_body, 0, unroll=False)

            for b in range(2):
                pltpu.make_async_copy(
                    table_hbm.at[idx_ref(0)], gbufs[b], gsems[b]).wait()
                for s in range(SUB):
                    pltpu.make_async_copy(
                        sbufs[b].at[pl.ds(s * LANES, LANES)],
                        out_hbm.at[scat_idx(0, s)], ssems[b]).wait()

    return emb_kernel


_EMB_KERNEL = None


def kernel(x, wte, pos_emb):
    global _EMB_KERNEL
    if _EMB_KERNEL is None:
        _EMB_KERNEL = _build_kernel()
    seq_len = x.shape[1]
    xt = x.astype(jnp.int32).T.reshape(NUM_WORKERS, POS_PER_W * BATCH)
    wte_b = lax.bitcast_convert_type(
        wte[:, _PERM].astype(jnp.bfloat16).reshape(VOCAB, DIM // 2, 2),
        jnp.int32)
    pos = pos_emb[:seq_len, :].astype(jnp.float32)
    out = _EMB_KERNEL(xt, wte_b, pos)
    return out.reshape(BATCH, SEQ, DIM)


# speedup vs baseline: 1.5048x; 1.5048x over previous
import functools

import jax
import jax.numpy as jnp
import numpy as np
from jax import lax
from jax.experimental import pallas as pl
from jax.experimental.pallas import tpu as pltpu
from jax.experimental.pallas import tpu_sc as plsc

VOCAB = 28996
DIM = 768
SEQ = 64
BATCH = 4096

NUM_CORES = 2
NUM_SUBCORES = 16
NUM_WORKERS = NUM_CORES * NUM_SUBCORES

B_TOTAL = BATCH * SEQ
POS_PER_W = SEQ // NUM_WORKERS
CHUNK = 32
N_CHUNKS = BATCH // CHUNK
LANES = 16
COLS = DIM // LANES
PAIRS = DIM // (2 * LANES)
SUB = CHUNK // LANES

_blk = np.arange(LANES)
_PERM = np.concatenate(
    [np.stack([_blk, _blk + LANES], axis=1).reshape(-1) + 2 * LANES * c
     for c in range(PAIRS)])


def _build_kernel():
    mesh = plsc.VectorSubcoreMesh(core_axis_name="c", subcore_axis_name="s")

    @functools.partial(
        pl.kernel,
        mesh=mesh,
        out_type=jax.ShapeDtypeStruct((B_TOTAL, DIM), jnp.float32),
        scratch_types=[
            pltpu.VMEM((POS_PER_W * BATCH,), jnp.int32),
            pltpu.VMEM((POS_PER_W, DIM), jnp.float32),
            pltpu.VMEM((CHUNK, DIM // 2), jnp.int32),
            pltpu.VMEM((CHUNK, DIM // 2), jnp.int32),
            pltpu.VMEM((CHUNK, DIM), jnp.float32),
            pltpu.VMEM((CHUNK, DIM), jnp.float32),
            pltpu.SemaphoreType.DMA,
            pltpu.SemaphoreType.DMA,
            pltpu.SemaphoreType.DMA,
            pltpu.SemaphoreType.DMA,
        ],
    )
    def emb_kernel(x_hbm, table_hbm, pos_hbm, out_hbm,
                   idx_v, pos_v, gbuf0, gbuf1, sbuf0, sbuf1,
                   gsem0, gsem1, ssem0, ssem1):
        wid = lax.axis_index("s") * NUM_CORES + lax.axis_index("c")
        pltpu.sync_copy(x_hbm.at[wid], idx_v)
        pltpu.sync_copy(pos_hbm.at[pl.ds(POS_PER_W * wid, POS_PER_W)], pos_v)

        gbufs = (gbuf0, gbuf1)
        sbufs = (sbuf0, sbuf1)
        gsems = (gsem0, gsem1)
        ssems = (ssem0, ssem1)
        iota = lax.iota(jnp.int32, LANES)

        for l in range(POS_PER_W):
            labs = POS_PER_W * wid + l

            def idx_ref(j, _l=l):
                return idx_v.at[pl.ds(_l * BATCH + j * CHUNK, CHUNK)]

            def scat_idx(j, s, _labs=labs):
                return iota * SEQ + ((j * CHUNK + s * LANES) * SEQ + _labs)

            pvecs = [pos_v[l, pl.ds(c * LANES, LANES)] for c in range(COLS)]

            pltpu.async_copy(table_hbm.at[idx_ref(0)], gbufs[0], gsems[0])
            pltpu.async_copy(table_hbm.at[idx_ref(1)], gbufs[1], gsems[1])

            def pair_body(jo, carry):
                for b in range(2):
                    j = 2 * jo + b
                    gbuf, sbuf = gbufs[b], sbufs[b]
                    gsem, ssem = gsems[b], ssems[b]

                    pltpu.make_async_copy(
                        table_hbm.at[idx_ref(j)], gbuf, gsem).wait()

                    @pl.when(jo >= 1)
                    def _():
                        for s in range(SUB):
                            pltpu.make_async_copy(
                                sbuf.at[pl.ds(s * LANES, LANES)],
                                out_hbm.at[scat_idx(j, s)], ssem).wait()

                    def row_body(r, c2):
                        for c in range(PAIRS):
                            w = gbuf[r, pl.ds(c * LANES, LANES)]
                            lo = lax.bitcast_convert_type(
                                w << 16, jnp.float32)
                            hi = lax.bitcast_convert_type(
                                w & jnp.int32(-65536), jnp.float32)
                            sl_lo = pl.ds(2 * c * LANES, LANES)
                            sl_hi = pl.ds((2 * c + 1) * LANES, LANES)
                            sbuf[r, sl_lo] = lo + pvecs[2 * c]
                            sbuf[r, sl_hi] = hi + pvecs[2 * c + 1]
                        return c2

                    lax.fori_loop(0, CHUNK, row_body, 0, unroll=False)

                    jn = jnp.minimum(j + 2, N_CHUNKS - 2 + b)
                    pltpu.async_copy(table_hbm.at[idx_ref(jn)], gbuf, gsem)

                    for s in range(SUB):
                        pltpu.async_copy(
                            sbuf.at[pl.ds(s * LANES, LANES)],
                            out_hbm.at[scat_idx(j, s)], ssem)
                return carry

            lax.fori_loop(0, N_CHUNKS // 2, pair_body, 0, unroll=False)

            for b in range(2):
                pltpu.make_async_copy(
                    table_hbm.at[idx_ref(0)], gbufs[b], gsems[b]).wait()
                for s in range(SUB):
                    pltpu.make_async_copy(
                        sbufs[b].at[pl.ds(s * LANES, LANES)],
                        out_hbm.at[scat_idx(0, s)], ssems[b]).wait()

    return emb_kernel


_EMB_KERNEL = None


def kernel(x, wte, pos_emb):
    global _EMB_KERNEL
    if _EMB_KERNEL is None:
        _EMB_KERNEL = _build_kernel()
    seq_len = x.shape[1]
    xt = x.astype(jnp.int32).T.reshape(NUM_WORKERS, POS_PER_W * BATCH)
    wte_b = lax.bitcast_convert_type(
        wte.reshape(VOCAB, PAIRS, 2, LANES)
        .swapaxes(2, 3)
        .astype(jnp.bfloat16)
        .reshape(VOCAB, DIM // 2, 2),
        jnp.int32)
    pos = pos_emb[:seq_len, :].astype(jnp.float32)
    out = _EMB_KERNEL(xt, wte_b, pos)
    return out.reshape(BATCH, SEQ, DIM)


# speedup vs baseline: 3.6679x; 2.4375x over previous
import functools

import jax
import jax.numpy as jnp
from jax import lax
from jax.experimental import pallas as pl
from jax.experimental.pallas import tpu as pltpu
from jax.experimental.pallas import tpu_sc as plsc

VOCAB = 28996
DIM = 768
SEQ = 64
BATCH = 4096

NUM_CORES = 2
NUM_SUBCORES = 16
NUM_WORKERS = NUM_CORES * NUM_SUBCORES

B_TOTAL = BATCH * SEQ
POS_PER_W = SEQ // NUM_WORKERS
CHUNK = 32
N_CHUNKS = BATCH // CHUNK
LANES = 16
COLS = DIM // LANES
SUB = CHUNK // LANES


def _build_kernel():
    mesh = plsc.VectorSubcoreMesh(core_axis_name="c", subcore_axis_name="s")

    @functools.partial(
        pl.kernel,
        mesh=mesh,
        out_type=jax.ShapeDtypeStruct((B_TOTAL, DIM), jnp.float32),
        scratch_types=[
            pltpu.VMEM((POS_PER_W * BATCH,), jnp.int32),
            pltpu.VMEM((POS_PER_W, DIM), jnp.float32),
            pltpu.VMEM((CHUNK, DIM), jnp.float32),
            pltpu.VMEM((CHUNK, DIM), jnp.float32),
            pltpu.VMEM((CHUNK, DIM), jnp.float32),
            pltpu.VMEM((CHUNK, DIM), jnp.float32),
            pltpu.SemaphoreType.DMA,
            pltpu.SemaphoreType.DMA,
            pltpu.SemaphoreType.DMA,
            pltpu.SemaphoreType.DMA,
        ],
    )
    def emb_kernel(x_hbm, table_hbm, pos_hbm, out_hbm,
                   idx_v, pos_v, gbuf0, gbuf1, sbuf0, sbuf1,
                   gsem0, gsem1, ssem0, ssem1):
        wid = lax.axis_index("s") * NUM_CORES + lax.axis_index("c")
        pltpu.sync_copy(x_hbm.at[wid], idx_v)
        pltpu.sync_copy(pos_hbm.at[pl.ds(POS_PER_W * wid, POS_PER_W)], pos_v)

        gbufs = (gbuf0, gbuf1)
        sbufs = (sbuf0, sbuf1)
        gsems = (gsem0, gsem1)
        ssems = (ssem0, ssem1)
        iota = lax.iota(jnp.int32, LANES)

        for l in range(POS_PER_W):
            labs = POS_PER_W * wid + l

            def idx_ref(j, _l=l):
                return idx_v.at[pl.ds(_l * BATCH + j * CHUNK, CHUNK)]

            def scat_idx(j, s, _labs=labs):
                return iota * SEQ + ((j * CHUNK + s * LANES) * SEQ + _labs)

            pvecs = [pos_v[l, pl.ds(c * LANES, LANES)] for c in range(COLS)]

            pltpu.async_copy(table_hbm.at[idx_ref(0)], gbufs[0], gsems[0])
            pltpu.async_copy(table_hbm.at[idx_ref(1)], gbufs[1], gsems[1])

            def pair_body(jo, carry):
                for b in range(2):
                    j = 2 * jo + b
                    gbuf, sbuf = gbufs[b], sbufs[b]
                    gsem, ssem = gsems[b], ssems[b]

                    pltpu.make_async_copy(
                        table_hbm.at[idx_ref(j)], gbuf, gsem).wait()

                    @pl.when(jo >= 1)
                    def _():
                        for s in range(SUB):
                            pltpu.make_async_copy(
                                sbuf.at[pl.ds(s * LANES, LANES)],
                                out_hbm.at[scat_idx(j, s)], ssem).wait()

                    def row_body(r, c2):
                        for c in range(COLS):
                            sl = pl.ds(c * LANES, LANES)
                            sbuf[r, sl] = gbuf[r, sl] + pvecs[c]
                        return c2

                    lax.fori_loop(0, CHUNK, row_body, 0, unroll=False)

                    jn = jnp.minimum(j + 2, N_CHUNKS - 2 + b)
                    pltpu.async_copy(table_hbm.at[idx_ref(jn)], gbuf, gsem)

                    for s in range(SUB):
                        pltpu.async_copy(
                            sbuf.at[pl.ds(s * LANES, LANES)],
                            out_hbm.at[scat_idx(j, s)], ssem)
                return carry

            lax.fori_loop(0, N_CHUNKS // 2, pair_body, 0, unroll=False)

            for b in range(2):
                pltpu.make_async_copy(
                    table_hbm.at[idx_ref(0)], gbufs[b], gsems[b]).wait()
                for s in range(SUB):
                    pltpu.make_async_copy(
                        sbufs[b].at[pl.ds(s * LANES, LANES)],
                        out_hbm.at[scat_idx(0, s)], ssems[b]).wait()

    return emb_kernel


_EMB_KERNEL = None


def kernel(x, wte, pos_emb):
    global _EMB_KERNEL
    if _EMB_KERNEL is None:
        _EMB_KERNEL = _build_kernel()
    seq_len = x.shape[1]
    xt = x.astype(jnp.int32).T.reshape(NUM_WORKERS, POS_PER_W * BATCH)
    pos = pos_emb[:seq_len, :].astype(jnp.float32)
    out = _EMB_KERNEL(xt, wte, pos)
    return out.reshape(BATCH, SEQ, DIM)
